# baseline (speedup 1.0000x reference)
import functools

import jax
import jax.numpy as jnp
from jax import lax
from jax.experimental import pallas as pl
from jax.experimental.pallas import tpu as pltpu
from jax.experimental.pallas import tpu_sc as plsc

_N = 10000
_E = 320000
_D = 128
_CH = 128
_NSUB = 16
_NCORE = 2
_NPAD = 10240
_ECH = 2560
_ECHR = _E // _CH
_JMAX = (_ECHR + _NSUB - 1) // _NSUB

_f32 = jnp.float32


def _sc_mesh():
    return plsc.VectorSubcoreMesh(core_axis_name="c", subcore_axis_name="s")


def _deg_kernel(dst_hbm, perm_hbm, out_a, out_b, outp_a, outp_b,
                acc_sh, dbuf, ones_v, zero_v, pidx0, pidx1, vals0, vals1,
                pidx_t, vals_t, sip0, sip1):
    c = lax.axis_index("c")
    s = lax.axis_index("s")
    pidx = [pidx0, pidx1]
    vals = [vals0, vals1]
    sip = [sip0, sip1]

    ones = jnp.full((16,), 1.0, dtype=_f32)
    for i in range(8):
        ones_v[pl.ds(i * 16, 16)] = ones
    zeros = jnp.zeros((16,), dtype=_f32)
    for i in range(40):
        zero_v[pl.ds(i * 16, 16)] = zeros
    pltpu.sync_copy(zero_v, acc_sh.at[pl.ds(s * 640, 640)])

    half = _ECH // _NCORE
    rows = half // _NSUB
    base = c * half + s * rows
    pltpu.sync_copy(dst_hbm.at[pl.ds(base, rows)], dbuf)
    plsc.subcore_barrier()

    def body(j, carry):
        pltpu.sync_copy(ones_v, acc_sh.at[dbuf.at[j]], add=True)
        return carry

    lax.fori_loop(0, rows, body, 0)
    plsc.subcore_barrier()

    @pl.when(c == 0)
    def _():
        pltpu.sync_copy(acc_sh.at[pl.ds(s * 640, 640)], out_a.at[pl.ds(s * 640, 640)])

    @pl.when(c == 1)
    def _():
        pltpu.sync_copy(acc_sh.at[pl.ds(s * 640, 640)], out_b.at[pl.ds(s * 640, 640)])

    nfull = _N // _CH

    def load_pair(j, k):
        nc = j * _NSUB + s

        @pl.when(nc < nfull)
        def _():
            pltpu.async_copy(perm_hbm.at[pl.ds(nc * _CH, _CH)], pidx[k], sip[k])

    def wait_scatter(j, k):
        nc = j * _NSUB + s

        @pl.when(nc < nfull)
        def _():
            pltpu.make_async_copy(perm_hbm.at[pl.ds(0, _CH)], pidx[k],
                                  sip[k]).wait()
            pltpu.sync_copy(acc_sh.at[pl.ds(nc * _CH, _CH)], vals[k])

            @pl.when(c == 0)
            def _():
                pltpu.sync_copy(vals[k], outp_a.at[pidx[k]])

            @pl.when(c == 1)
            def _():
                pltpu.sync_copy(vals[k], outp_b.at[pidx[k]])

    load_pair(0, 0)
    load_pair(1, 1)

    def pbody(g, carry):
        j0 = 2 * g
        wait_scatter(j0, 0)
        load_pair(j0 + 2, 0)
        wait_scatter(j0 + 1, 1)
        load_pair(j0 + 3, 1)
        return carry

    lax.fori_loop(0, ((nfull + _NSUB - 1) // _NSUB + 1) // 2, pbody, 0)

    tail = _N - nfull * _CH

    @pl.when(s == 0)
    def _():
        pltpu.sync_copy(perm_hbm.at[pl.ds(nfull * _CH, tail)], pidx_t)
        pltpu.sync_copy(acc_sh.at[pl.ds(nfull * _CH, tail)], vals_t)

        @pl.when(c == 0)
        def _():
            pltpu.sync_copy(vals_t, outp_a.at[pidx_t])

        @pl.when(c == 1)
        def _():
            pltpu.sync_copy(vals_t, outp_b.at[pidx_t])


def _sc_deg(dst2d, perm):
    kern = pl.kernel(
        _deg_kernel,
        mesh=_sc_mesh(),
        out_type=tuple(jax.ShapeDtypeStruct((_NPAD,), _f32) for _ in range(4)),
        scratch_types=[
            pltpu.VMEM_SHARED((_NPAD,), _f32),
            pltpu.VMEM((_ECH // _NCORE // _NSUB, _CH), jnp.int32),
            pltpu.VMEM((_CH,), _f32),
            pltpu.VMEM((640,), _f32),
            pltpu.VMEM((_CH,), jnp.int32),
            pltpu.VMEM((_CH,), jnp.int32),
            pltpu.VMEM((_CH,), _f32),
            pltpu.VMEM((_CH,), _f32),
            pltpu.VMEM((16,), jnp.int32),
            pltpu.VMEM((16,), _f32),
            pltpu.SemaphoreType.DMA,
            pltpu.SemaphoreType.DMA,
        ],
    )
    return kern(dst2d, perm)


_RCH = 80
_NRCH = _N // _RCH


def _agg_half(tbl, out, src_hbm, dst_hbm, acc_sh, isrc, idst, bufs, sgs, sis, s,
              perm_hbm=None, isp=None, sip=None, pinit=None, pinit_t=None):
    if perm_hbm is None:
        def init_body(j, carry):
            cid = j * _NSUB + s

            @pl.when(cid < _NRCH)
            def _():
                pltpu.sync_copy(tbl.at[pl.ds(cid * _RCH, _RCH)],
                                acc_sh.at[pl.ds(cid * _RCH, _RCH)])

            return carry

        lax.fori_loop(0, (_NRCH + _NSUB - 1) // _NSUB, init_body, 0)
    else:
        nfull = _N // _CH

        def init_body(j, carry):
            nc = j * _NSUB + s

            @pl.when(nc < nfull)
            def _():
                pltpu.sync_copy(perm_hbm.at[pl.ds(nc * _CH, _CH)], pinit)
                pltpu.async_copy(tbl.at[pinit], bufs[0], sgs[0]).wait()
                pltpu.sync_copy(bufs[0], acc_sh.at[pl.ds(nc * _CH, _CH)])

            return carry

        lax.fori_loop(0, (nfull + _NSUB - 1) // _NSUB, init_body, 0)
        tail = _N - nfull * _CH

        @pl.when(s == 0)
        def _():
            pltpu.sync_copy(perm_hbm.at[pl.ds(nfull * _CH, tail)], pinit_t)
            pltpu.async_copy(tbl.at[pinit_t], bufs[0].at[pl.ds(0, tail)],
                             sgs[0]).wait()
            pltpu.sync_copy(bufs[0].at[pl.ds(0, tail)],
                            acc_sh.at[pl.ds(nfull * _CH, tail)])

    plsc.subcore_barrier()

    def load_idx(j, k):
        cid = j * _NSUB + s

        @pl.when(cid < _ECHR)
        def _():
            pltpu.async_copy(src_hbm.at[pl.ds(cid * _CH, _CH)], isrc[k], sis[k])
            pltpu.async_copy(dst_hbm.at[pl.ds(cid * _CH, _CH)], idst[k], sis[k])

    def wait_idx(j, k):
        cid = j * _NSUB + s

        @pl.when(cid < _ECHR)
        def _():
            pltpu.make_async_copy(src_hbm.at[pl.ds(0, _CH)], isrc[k], sis[k]).wait()
            pltpu.make_async_copy(src_hbm.at[pl.ds(0, _CH)], idst[k], sis[k]).wait()

    def fire_pres(j, k):
        cid = j * _NSUB + s

        @pl.when(cid < _ECHR)
        def _():
            pltpu.async_copy(perm_hbm.at[isrc[k]], isp[k], sip[k])

    def wait_pres(j, k):
        cid = j * _NSUB + s

        @pl.when(cid < _ECHR)
        def _():
            pltpu.make_async_copy(src_hbm.at[pl.ds(0, _CH)], isp[k], sip[k]).wait()

    def fire_gather(j, k, b):
        cid = j * _NSUB + s
        gidx = isrc[k] if perm_hbm is None else isp[k]

        @pl.when(cid < _ECHR)
        def _():
            pltpu.async_copy(tbl.at[gidx], bufs[b], sgs[b])

    def drain_scatter(j, k, b):
        cid = j * _NSUB + s

        @pl.when(cid < _ECHR)
        def _():
            pltpu.make_async_copy(tbl.at[pl.ds(0, _CH)], bufs[b], sgs[b]).wait()
            pltpu.sync_copy(bufs[b], acc_sh.at[idst[k]], add=True)

    if perm_hbm is None:
        load_idx(0, 0)
        load_idx(1, 1)
        wait_idx(0, 0)
        fire_gather(0, 0, 0)

        def body(u, carry):
            j = u * 4
            for k in range(4):
                jj = j + k
                b = k % 2
                load_idx(jj + 2, (k + 2) % 4)
                wait_idx(jj + 1, (k + 1) % 4)
                fire_gather(jj + 1, (k + 1) % 4, 1 - b)
                drain_scatter(jj, k, b)
            return carry
    else:
        load_idx(0, 0)
        load_idx(1, 1)
        load_idx(2, 2)
        wait_idx(0, 0)
        fire_pres(0, 0)
        wait_idx(1, 1)
        fire_pres(1, 1)
        wait_pres(0, 0)
        fire_gather(0, 0, 0)

        def body(u, carry):
            j = u * 4
            for k in range(4):
                jj = j + k
                b = k % 2
                load_idx(jj + 3, (k + 3) % 4)
                wait_idx(jj + 2, (k + 2) % 4)
                fire_pres(jj + 2, (k + 2) % 4)
                wait_pres(jj + 1, (k + 1) % 4)
                fire_gather(jj + 1, (k + 1) % 4, 1 - b)
                drain_scatter(jj, k, b)
            return carry

    lax.fori_loop(0, (_JMAX + 3) // 4, body, 0)
    plsc.subcore_barrier()

    def wb_body(j, carry):
        cid = j * _NSUB + s

        @pl.when(cid < _NRCH)
        def _():
            pltpu.sync_copy(acc_sh.at[pl.ds(cid * _RCH, _RCH)],
                            out.at[pl.ds(cid * _RCH, _RCH)])

        return carry

    lax.fori_loop(0, (_NRCH + _NSUB - 1) // _NSUB, wb_body, 0)


def _make_agg_kernel(with_perm):
    def _agg_kernel(tbl_a, tbl_b, src_hbm, dst_hbm, perm_hbm, out_a, out_b,
                    acc_sh,
                    is0, is1, is2, is3, id0, id1, id2, id3,
                    ip0, ip1, ip2, ip3, pinit, pinit_t, buf_a, buf_b,
                    sg_a, sg_b, si0, si1, si2, si3, sp0, sp1, sp2, sp3):
        c = lax.axis_index("c")
        s = lax.axis_index("s")
        isrc = [is0, is1, is2, is3]
        idst = [id0, id1, id2, id3]
        bufs = [buf_a, buf_b]
        sgs = [sg_a, sg_b]
        sis = [si0, si1, si2, si3]
        isp = [ip0, ip1, ip2, ip3]
        sip = [sp0, sp1, sp2, sp3]

        @pl.when(c == 0)
        def _():
            _agg_half(tbl_a, out_a, src_hbm, dst_hbm, acc_sh,
                      isrc, idst, bufs, sgs, sis, s)

        @pl.when(c == 1)
        def _():
            if with_perm:
                _agg_half(tbl_b, out_b, src_hbm, dst_hbm, acc_sh,
                          isrc, idst, bufs, sgs, sis, s,
                          perm_hbm=perm_hbm, isp=isp, sip=sip,
                          pinit=pinit, pinit_t=pinit_t)
            else:
                _agg_half(tbl_b, out_b, src_hbm, dst_hbm, acc_sh,
                          isrc, idst, bufs, sgs, sis, s)

    return _agg_kernel


def _sc_agg(tbl_a, tbl_b, src, dst, perm, with_perm):
    kern = pl.kernel(
        _make_agg_kernel(with_perm),
        mesh=_sc_mesh(),
        out_type=(
            jax.ShapeDtypeStruct((_N, _D), _f32),
            jax.ShapeDtypeStruct((_N, _D), _f32),
        ),
        scratch_types=(
            [pltpu.VMEM_SHARED((_N, _D), _f32)]
            + [pltpu.VMEM((_CH,), jnp.int32) for _ in range(12)]
            + [pltpu.VMEM((_CH,), jnp.int32), pltpu.VMEM((16,), jnp.int32)]
            + [pltpu.VMEM((_CH, _D), _f32) for _ in range(2)]
            + [pltpu.SemaphoreType.DMA for _ in range(10)]
        ),
    )
    return kern(tbl_a, tbl_b, src, dst, perm)


_BR = 1000


def _dinv(pa, pb):
    return lax.rsqrt(pa + pb + 1.0)


def _mm_scale_body(x_ref, w_ref, pa_ref, pb_ref, ppa_ref, ppb_ref,
                   oa_ref, ob_ref):
    h = jnp.dot(x_ref[...], w_ref[...], preferred_element_type=_f32)
    oa_ref[...] = h * _dinv(pa_ref[...], pb_ref[...])
    ob_ref[...] = h * _dinv(ppa_ref[...], ppb_ref[...])


def _tc_mm_scale(x, W, pa, pb, ppa, ppb):
    return pl.pallas_call(
        _mm_scale_body,
        grid=(_N // _BR,),
        in_specs=[
            pl.BlockSpec((_BR, _D), lambda i: (i, 0)),
            pl.BlockSpec((_D, _D), lambda i: (0, 0)),
            pl.BlockSpec((_BR, 1), lambda i: (i, 0)),
            pl.BlockSpec((_BR, 1), lambda i: (i, 0)),
            pl.BlockSpec((_BR, 1), lambda i: (i, 0)),
            pl.BlockSpec((_BR, 1), lambda i: (i, 0)),
        ],
        out_specs=[
            pl.BlockSpec((_BR, _D), lambda i: (i, 0)),
            pl.BlockSpec((_BR, _D), lambda i: (i, 0)),
        ],
        out_shape=[
            jax.ShapeDtypeStruct((_N, _D), _f32),
            jax.ShapeDtypeStruct((_N, _D), _f32),
        ],
    )(x, W, pa, pb, ppa, ppb)


def _layer2_body(sa_ref, sb_ref, pa_ref, pb_ref, b1_ref, w2_ref, oa_ref, ob_ref):
    d = _dinv(pa_ref[...], pb_ref[...])
    za = jnp.maximum(sa_ref[...] * d + b1_ref[...], 0.0)
    zb = jnp.maximum(sb_ref[...] * d + b1_ref[...], 0.0)
    oa_ref[...] = jnp.dot(za, w2_ref[...], preferred_element_type=_f32) * d
    ob_ref[...] = jnp.dot(zb, w2_ref[...], preferred_element_type=_f32) * d


def _tc_layer2(sa, sb, pa, pb, b1, W2):
    return pl.pallas_call(
        _layer2_body,
        grid=(_N // _BR,),
        in_specs=[
            pl.BlockSpec((_BR, _D), lambda i: (i, 0)),
            pl.BlockSpec((_BR, _D), lambda i: (i, 0)),
            pl.BlockSpec((_BR, 1), lambda i: (i, 0)),
            pl.BlockSpec((_BR, 1), lambda i: (i, 0)),
            pl.BlockSpec((1, _D), lambda i: (0, 0)),
            pl.BlockSpec((_D, _D), lambda i: (0, 0)),
        ],
        out_specs=[
            pl.BlockSpec((_BR, _D), lambda i: (i, 0)),
            pl.BlockSpec((_BR, _D), lambda i: (i, 0)),
        ],
        out_shape=[
            jax.ShapeDtypeStruct((_N, _D), _f32),
            jax.ShapeDtypeStruct((_N, _D), _f32),
        ],
    )(sa, sb, pa, pb, b1, W2)


def _readout_body(sa_ref, sb_ref, pa_ref, pb_ref, b2_ref, wb_ref, bb_ref,
                  pos_ref, neg_ref, colsum, vrow, c0):
    p = pl.program_id(0)
    j = pl.program_id(1)
    d = _dinv(pa_ref[...], pb_ref[...])

    @pl.when(p == 0)
    def _():
        @pl.when(j == 0)
        def _():
            colsum[...] = jnp.zeros_like(colsum)

        ha = sa_ref[...] * d
        colsum[...] += jnp.sum(ha, axis=0, keepdims=True)

    @pl.when(p == 1)
    def _():
        @pl.when(j == 0)
        def _():
            mean = colsum[...] * (1.0 / _N) + b2_ref[...]
            srow = 1.0 / (1.0 + jnp.exp(-mean))
            v = lax.dot_general(srow, wb_ref[...], (((1,), (1,)), ((), ())),
                                preferred_element_type=_f32)
            vrow[...] = v
            c0[...] = jnp.sum(b2_ref[...] * v, axis=1, keepdims=True) + bb_ref[...]

        ha = sa_ref[...] * d
        hb = sb_ref[...] * d
        v = vrow[...]
        pos_ref[...] = jnp.sum(ha * v, axis=1, keepdims=True) + c0[...]
        neg_ref[...] = jnp.sum(hb * v, axis=1, keepdims=True) + c0[...]


def _tc_readout(sa, sb, pa, pb, b2, wb, bb):
    return pl.pallas_call(
        _readout_body,
        grid=(2, _N // _BR),
        in_specs=[
            pl.BlockSpec((_BR, _D), lambda p, j: (j, 0)),
            pl.BlockSpec((_BR, _D), lambda p, j: (j, 0)),
            pl.BlockSpec((_BR, 1), lambda p, j: (j, 0)),
            pl.BlockSpec((_BR, 1), lambda p, j: (j, 0)),
            pl.BlockSpec((1, _D), lambda p, j: (0, 0)),
            pl.BlockSpec((_D, _D), lambda p, j: (0, 0)),
            pl.BlockSpec((1, 1), lambda p, j: (0, 0)),
        ],
        out_specs=[
            pl.BlockSpec((_BR, 1), lambda p, j: (j, 0)),
            pl.BlockSpec((_BR, 1), lambda p, j: (j, 0)),
        ],
        out_shape=[
            jax.ShapeDtypeStruct((_N, 1), _f32),
            jax.ShapeDtypeStruct((_N, 1), _f32),
        ],
        scratch_shapes=[
            pltpu.VMEM((1, _D), _f32),
            pltpu.VMEM((1, _D), _f32),
            pltpu.VMEM((1, 1), _f32),
        ],
    )(sa, sb, pa, pb, b2, wb, bb)


def kernel(x, edge_index, W1, b1, W2, b2, Wb, bb, perm):
    src = edge_index[0]
    dst = edge_index[1]
    perm = perm.astype(jnp.int32)

    npad_e = _ECH * _CH - _E
    dst2d = jnp.concatenate([dst, jnp.full((npad_e,), _N, jnp.int32)]).reshape(_ECH, _CH)

    dega, degb, degpa, degpb = _sc_deg(dst2d, perm)
    pa = dega[:_N].reshape(_N, 1)
    pb = degb[:_N].reshape(_N, 1)
    ppa = degpa[:_N].reshape(_N, 1)
    ppb = degpb[:_N].reshape(_N, 1)

    p1a, qb = _tc_mm_scale(x, W1, pa, pb, ppa, ppb)
    s1a, s1b = _sc_agg(p1a, qb, src, dst, perm, with_perm=True)

    p2a, p2b = _tc_layer2(s1a, s1b, pa, pb, b1.reshape(1, _D), W2)
    s2a, s2b = _sc_agg(p2a, p2b, src, dst, perm, with_perm=False)

    pos, neg = _tc_readout(s2a, s2b, pa, pb, b2.reshape(1, _D),
                           Wb.reshape(_D, _D), bb.reshape(1, 1))
    return (pos, neg)


# speedup vs baseline: 1.1276x; 1.1276x over previous
import functools

import jax
import jax.numpy as jnp
from jax import lax
from jax.experimental import pallas as pl
from jax.experimental.pallas import tpu as pltpu
from jax.experimental.pallas import tpu_sc as plsc

_N = 10000
_E = 320000
_D = 128
_CH = 128
_NSUB = 16
_NCORE = 2
_NPAD = 10240
_ECH = 2560
_ECHR = _E // _CH
_JMAX = (_ECHR + _NSUB - 1) // _NSUB

_f32 = jnp.float32


def _sc_mesh():
    return plsc.VectorSubcoreMesh(core_axis_name="c", subcore_axis_name="s")


def _deg_kernel(dst_hbm, out_a, out_b, acc_sh, dbuf, ones_v, zero_v):
    c = lax.axis_index("c")
    s = lax.axis_index("s")

    ones = jnp.full((16,), 1.0, dtype=_f32)
    for i in range(8):
        ones_v[pl.ds(i * 16, 16)] = ones
    zeros = jnp.zeros((16,), dtype=_f32)
    for i in range(40):
        zero_v[pl.ds(i * 16, 16)] = zeros
    pltpu.sync_copy(zero_v, acc_sh.at[pl.ds(s * 640, 640)])

    half = _ECH // _NCORE
    rows = half // _NSUB
    base = c * half + s * rows
    pltpu.sync_copy(dst_hbm.at[pl.ds(base, rows)], dbuf)
    plsc.subcore_barrier()

    def body(j, carry):
        pltpu.sync_copy(ones_v, acc_sh.at[dbuf.at[j]], add=True)
        return carry

    lax.fori_loop(0, rows, body, 0)
    plsc.subcore_barrier()

    @pl.when(c == 0)
    def _():
        pltpu.sync_copy(acc_sh.at[pl.ds(s * 640, 640)], out_a.at[pl.ds(s * 640, 640)])

    @pl.when(c == 1)
    def _():
        pltpu.sync_copy(acc_sh.at[pl.ds(s * 640, 640)], out_b.at[pl.ds(s * 640, 640)])


def _sc_deg(dst2d):
    kern = pl.kernel(
        _deg_kernel,
        mesh=_sc_mesh(),
        out_type=(
            jax.ShapeDtypeStruct((_NPAD,), _f32),
            jax.ShapeDtypeStruct((_NPAD,), _f32),
        ),
        scratch_types=[
            pltpu.VMEM_SHARED((_NPAD,), _f32),
            pltpu.VMEM((_ECH // _NCORE // _NSUB, _CH), jnp.int32),
            pltpu.VMEM((_CH,), _f32),
            pltpu.VMEM((640,), _f32),
        ],
    )
    return kern(dst2d)


def _perm_kernel(h1_hbm, perm_hbm, out_hbm, idx_v, rows_v, idx_t, rows_t, sem):
    c = lax.axis_index("c")
    s = lax.axis_index("s")
    w = s * _NCORE + c

    nfull = _N // _CH
    nw = _NCORE * _NSUB

    def body(j, carry):
        cid = j * nw + w

        @pl.when(cid < nfull)
        def _():
            pltpu.sync_copy(perm_hbm.at[pl.ds(cid * _CH, _CH)], idx_v)
            pltpu.async_copy(h1_hbm.at[idx_v], rows_v, sem).wait()
            pltpu.sync_copy(rows_v, out_hbm.at[pl.ds(cid * _CH, _CH)])

        return carry

    lax.fori_loop(0, (nfull + nw - 1) // nw, body, 0)

    tail = _N - nfull * _CH

    @pl.when(w == 0)
    def _():
        pltpu.sync_copy(perm_hbm.at[pl.ds(nfull * _CH, tail)], idx_t)
        pltpu.async_copy(h1_hbm.at[idx_t], rows_t, sem).wait()
        pltpu.sync_copy(rows_t, out_hbm.at[pl.ds(nfull * _CH, tail)])


def _sc_perm(h1, perm):
    kern = pl.kernel(
        _perm_kernel,
        mesh=_sc_mesh(),
        out_type=jax.ShapeDtypeStruct((_N, _D), _f32),
        scratch_types=[
            pltpu.VMEM((_CH,), jnp.int32),
            pltpu.VMEM((_CH, _D), _f32),
            pltpu.VMEM((16,), jnp.int32),
            pltpu.VMEM((16, _D), _f32),
            pltpu.SemaphoreType.DMA,
        ],
    )
    return kern(h1, perm)


_RCH = 80
_NRCH = _N // _RCH


def _agg_half(tbl, out, src_hbm, dst_hbm, acc_sh, isrc, idst, bufs, sgs, sis, s):
    def init_body(j, carry):
        cid = j * _NSUB + s

        @pl.when(cid < _NRCH)
        def _():
            pltpu.sync_copy(tbl.at[pl.ds(cid * _RCH, _RCH)],
                            acc_sh.at[pl.ds(cid * _RCH, _RCH)])

        return carry

    lax.fori_loop(0, (_NRCH + _NSUB - 1) // _NSUB, init_body, 0)
    plsc.subcore_barrier()

    def load_idx(j, k):
        cid = j * _NSUB + s

        @pl.when(cid < _ECHR)
        def _():
            pltpu.async_copy(src_hbm.at[pl.ds(cid * _CH, _CH)], isrc[k], sis[k])
            pltpu.async_copy(dst_hbm.at[pl.ds(cid * _CH, _CH)], idst[k], sis[k])

    def wait_idx(j, k):
        cid = j * _NSUB + s

        @pl.when(cid < _ECHR)
        def _():
            pltpu.make_async_copy(src_hbm.at[pl.ds(0, _CH)], isrc[k], sis[k]).wait()
            pltpu.make_async_copy(src_hbm.at[pl.ds(0, _CH)], idst[k], sis[k]).wait()

    def fire_gather(j, k, b):
        cid = j * _NSUB + s

        @pl.when(cid < _ECHR)
        def _():
            pltpu.async_copy(tbl.at[isrc[k]], bufs[b], sgs[b])

    def drain_scatter(j, k, b):
        cid = j * _NSUB + s

        @pl.when(cid < _ECHR)
        def _():
            pltpu.make_async_copy(tbl.at[pl.ds(0, _CH)], bufs[b], sgs[b]).wait()
            pltpu.sync_copy(bufs[b], acc_sh.at[idst[k]], add=True)

    load_idx(0, 0)
    load_idx(1, 1)
    wait_idx(0, 0)
    fire_gather(0, 0, 0)

    def body(u, carry):
        j = u * 4
        for k in range(4):
            jj = j + k
            b = k % 2
            load_idx(jj + 2, (k + 2) % 4)
            wait_idx(jj + 1, (k + 1) % 4)
            fire_gather(jj + 1, (k + 1) % 4, 1 - b)
            drain_scatter(jj, k, b)
        return carry

    lax.fori_loop(0, (_JMAX + 3) // 4, body, 0)
    plsc.subcore_barrier()

    def wb_body(j, carry):
        cid = j * _NSUB + s

        @pl.when(cid < _NRCH)
        def _():
            pltpu.sync_copy(acc_sh.at[pl.ds(cid * _RCH, _RCH)],
                            out.at[pl.ds(cid * _RCH, _RCH)])

        return carry

    lax.fori_loop(0, (_NRCH + _NSUB - 1) // _NSUB, wb_body, 0)


def _agg_kernel(tbl_a, tbl_b, src_hbm, dst_hbm, out_a, out_b, acc_sh,
                is0, is1, is2, is3, id0, id1, id2, id3, buf_a, buf_b,
                sg_a, sg_b, si0, si1, si2, si3):
    c = lax.axis_index("c")
    s = lax.axis_index("s")
    isrc = [is0, is1, is2, is3]
    idst = [id0, id1, id2, id3]
    bufs = [buf_a, buf_b]
    sgs = [sg_a, sg_b]
    sis = [si0, si1, si2, si3]

    @pl.when(c == 0)
    def _():
        _agg_half(tbl_a, out_a, src_hbm, dst_hbm, acc_sh,
                  isrc, idst, bufs, sgs, sis, s)

    @pl.when(c == 1)
    def _():
        _agg_half(tbl_b, out_b, src_hbm, dst_hbm, acc_sh,
                  isrc, idst, bufs, sgs, sis, s)


def _sc_agg(tbl_a, tbl_b, src, dst):
    kern = pl.kernel(
        _agg_kernel,
        mesh=_sc_mesh(),
        out_type=(
            jax.ShapeDtypeStruct((_N, _D), _f32),
            jax.ShapeDtypeStruct((_N, _D), _f32),
        ),
        scratch_types=(
            [pltpu.VMEM_SHARED((_N, _D), _f32)]
            + [pltpu.VMEM((_CH,), jnp.int32) for _ in range(8)]
            + [pltpu.VMEM((_CH, _D), _f32) for _ in range(2)]
            + [pltpu.SemaphoreType.DMA for _ in range(6)]
        ),
    )
    return kern(tbl_a, tbl_b, src, dst)


_BR = 1000


def _dinv(pa, pb):
    return lax.rsqrt(pa + pb + 1.0)


def _mm_body(x_ref, w_ref, o_ref):
    o_ref[...] = jnp.dot(x_ref[...], w_ref[...], preferred_element_type=_f32)


def _tc_matmul(x, W):
    return pl.pallas_call(
        _mm_body,
        grid=(_N // _BR,),
        in_specs=[
            pl.BlockSpec((_BR, _D), lambda i: (i, 0)),
            pl.BlockSpec((_D, _D), lambda i: (0, 0)),
        ],
        out_specs=pl.BlockSpec((_BR, _D), lambda i: (i, 0)),
        out_shape=jax.ShapeDtypeStruct((_N, _D), _f32),
    )(x, W)


def _scale2_body(h_ref, hp_ref, pa_ref, pb_ref, oa_ref, ob_ref):
    d = _dinv(pa_ref[...], pb_ref[...])
    oa_ref[...] = h_ref[...] * d
    ob_ref[...] = hp_ref[...] * d


def _tc_scale2(h1, h1p, pa, pb):
    return pl.pallas_call(
        _scale2_body,
        grid=(_N // _BR,),
        in_specs=[
            pl.BlockSpec((_BR, _D), lambda i: (i, 0)),
            pl.BlockSpec((_BR, _D), lambda i: (i, 0)),
            pl.BlockSpec((_BR, 1), lambda i: (i, 0)),
            pl.BlockSpec((_BR, 1), lambda i: (i, 0)),
        ],
        out_specs=[
            pl.BlockSpec((_BR, _D), lambda i: (i, 0)),
            pl.BlockSpec((_BR, _D), lambda i: (i, 0)),
        ],
        out_shape=[
            jax.ShapeDtypeStruct((_N, _D), _f32),
            jax.ShapeDtypeStruct((_N, _D), _f32),
        ],
    )(h1, h1p, pa, pb)


def _layer2_body(sa_ref, sb_ref, pa_ref, pb_ref, b1_ref, w2_ref, oa_ref, ob_ref):
    d = _dinv(pa_ref[...], pb_ref[...])
    za = jnp.maximum(sa_ref[...] * d + b1_ref[...], 0.0)
    zb = jnp.maximum(sb_ref[...] * d + b1_ref[...], 0.0)
    oa_ref[...] = jnp.dot(za, w2_ref[...], preferred_element_type=_f32) * d
    ob_ref[...] = jnp.dot(zb, w2_ref[...], preferred_element_type=_f32) * d


def _tc_layer2(sa, sb, pa, pb, b1, W2):
    return pl.pallas_call(
        _layer2_body,
        grid=(_N // _BR,),
        in_specs=[
            pl.BlockSpec((_BR, _D), lambda i: (i, 0)),
            pl.BlockSpec((_BR, _D), lambda i: (i, 0)),
            pl.BlockSpec((_BR, 1), lambda i: (i, 0)),
            pl.BlockSpec((_BR, 1), lambda i: (i, 0)),
            pl.BlockSpec((1, _D), lambda i: (0, 0)),
            pl.BlockSpec((_D, _D), lambda i: (0, 0)),
        ],
        out_specs=[
            pl.BlockSpec((_BR, _D), lambda i: (i, 0)),
            pl.BlockSpec((_BR, _D), lambda i: (i, 0)),
        ],
        out_shape=[
            jax.ShapeDtypeStruct((_N, _D), _f32),
            jax.ShapeDtypeStruct((_N, _D), _f32),
        ],
    )(sa, sb, pa, pb, b1, W2)


def _readout_body(sa_ref, sb_ref, pa_ref, pb_ref, b2_ref, wb_ref, bb_ref,
                  pos_ref, neg_ref, colsum, vrow, c0):
    p = pl.program_id(0)
    j = pl.program_id(1)
    d = _dinv(pa_ref[...], pb_ref[...])

    @pl.when(p == 0)
    def _():
        @pl.when(j == 0)
        def _():
            colsum[...] = jnp.zeros_like(colsum)

        ha = sa_ref[...] * d
        colsum[...] += jnp.sum(ha, axis=0, keepdims=True)

    @pl.when(p == 1)
    def _():
        @pl.when(j == 0)
        def _():
            mean = colsum[...] * (1.0 / _N) + b2_ref[...]
            srow = 1.0 / (1.0 + jnp.exp(-mean))
            v = lax.dot_general(srow, wb_ref[...], (((1,), (1,)), ((), ())),
                                preferred_element_type=_f32)
            vrow[...] = v
            c0[...] = jnp.sum(b2_ref[...] * v, axis=1, keepdims=True) + bb_ref[...]

        ha = sa_ref[...] * d
        hb = sb_ref[...] * d
        v = vrow[...]
        pos_ref[...] = jnp.sum(ha * v, axis=1, keepdims=True) + c0[...]
        neg_ref[...] = jnp.sum(hb * v, axis=1, keepdims=True) + c0[...]


def _tc_readout(sa, sb, pa, pb, b2, wb, bb):
    return pl.pallas_call(
        _readout_body,
        grid=(2, _N // _BR),
        in_specs=[
            pl.BlockSpec((_BR, _D), lambda p, j: (j, 0)),
            pl.BlockSpec((_BR, _D), lambda p, j: (j, 0)),
            pl.BlockSpec((_BR, 1), lambda p, j: (j, 0)),
            pl.BlockSpec((_BR, 1), lambda p, j: (j, 0)),
            pl.BlockSpec((1, _D), lambda p, j: (0, 0)),
            pl.BlockSpec((_D, _D), lambda p, j: (0, 0)),
            pl.BlockSpec((1, 1), lambda p, j: (0, 0)),
        ],
        out_specs=[
            pl.BlockSpec((_BR, 1), lambda p, j: (j, 0)),
            pl.BlockSpec((_BR, 1), lambda p, j: (j, 0)),
        ],
        out_shape=[
            jax.ShapeDtypeStruct((_N, 1), _f32),
            jax.ShapeDtypeStruct((_N, 1), _f32),
        ],
        scratch_shapes=[
            pltpu.VMEM((1, _D), _f32),
            pltpu.VMEM((1, _D), _f32),
            pltpu.VMEM((1, 1), _f32),
        ],
    )(sa, sb, pa, pb, b2, wb, bb)


def kernel(x, edge_index, W1, b1, W2, b2, Wb, bb, perm):
    src = edge_index[0]
    dst = edge_index[1]
    perm = perm.astype(jnp.int32)

    npad_e = _ECH * _CH - _E
    dst2d = jnp.concatenate([dst, jnp.full((npad_e,), _N, jnp.int32)]).reshape(_ECH, _CH)

    dega, degb = _sc_deg(dst2d)
    pa = dega[:_N].reshape(_N, 1)
    pb = degb[:_N].reshape(_N, 1)

    h1 = _tc_matmul(x, W1)
    h1p = _sc_perm(h1, perm)
    p1a, p1b = _tc_scale2(h1, h1p, pa, pb)
    s1a, s1b = _sc_agg(p1a, p1b, src, dst)

    p2a, p2b = _tc_layer2(s1a, s1b, pa, pb, b1.reshape(1, _D), W2)
    s2a, s2b = _sc_agg(p2a, p2b, src, dst)

    pos, neg = _tc_readout(s2a, s2b, pa, pb, b2.reshape(1, _D),
                           Wb.reshape(_D, _D), bb.reshape(1, 1))
    return (pos, neg)
